# baseline (speedup 1.0000x reference)
import functools

import jax
import jax.numpy as jnp
from jax import lax
from jax.experimental import pallas as pl
from jax.experimental.pallas import tpu as pltpu
from jax.experimental.pallas import tpu_sc as plsc

D_M = 128
T_LEN = 200
N_WORKERS = 32
LANES = 16
CHUNKS_PER_T = D_M // LANES
RING = 4


def _sc_body(x_hbm, pe_hbm, out_hbm, pe_v, buf0, buf1, buf2, buf3,
             psem, is0, is1, is2, is3, os0, os1, os2, os3):
    nc = lax.axis_size("c")
    wid = lax.axis_index("s") * nc + lax.axis_index("c")
    b_per_w = out_hbm.shape[0] // N_WORKERS
    base = wid * b_per_w

    pe_h = pltpu.async_copy(pe_hbm, pe_v, psem)

    bufs = (buf0, buf1, buf2, buf3)
    isems = (is0, is1, is2, is3)
    osems = (os0, os1, os2, os3)
    in_h = [None] * RING
    out_h = [None] * RING

    for i in range(RING - 1):
        in_h[i] = pltpu.async_copy(x_hbm.at[base + i], bufs[i], isems[i])
    pe_h.wait()
    for i in range(b_per_w):
        sl = i % RING
        in_h[sl].wait()
        buf = bufs[sl]

        def add_body(t, _, buf=buf):
            for c in range(CHUNKS_PER_T):
                s = pl.ds(c * LANES, LANES)
                buf[t, s] = buf[t, s] + pe_v[t, s]
            return 0

        lax.fori_loop(0, T_LEN, add_body, 0)
        out_h[sl] = pltpu.async_copy(buf, out_hbm.at[base + i], osems[sl])
        nxt = i + RING - 1
        if nxt < b_per_w:
            nsl = nxt % RING
            if out_h[nsl] is not None:
                out_h[nsl].wait()
            in_h[nsl] = pltpu.async_copy(
                x_hbm.at[base + nxt], bufs[nsl], isems[nsl])
    for h in out_h:
        if h is not None:
            h.wait()


def _sc_add(x, pe_t):
    B = x.shape[0]
    mesh = plsc.VectorSubcoreMesh(core_axis_name="c", subcore_axis_name="s")
    f = functools.partial(
        pl.kernel,
        out_type=jax.ShapeDtypeStruct((B, T_LEN, D_M), jnp.float32),
        mesh=mesh,
        scratch_types=(
            [pltpu.VMEM((T_LEN, D_M), jnp.float32)] * (1 + RING)
            + [pltpu.SemaphoreType.DMA] * (1 + 2 * RING)
        ),
    )(_sc_body)
    return f(x, pe_t)


def kernel(x, pe):
    B, T, D = x.shape
    return _sc_add(x, pe[:T])


# speedup vs baseline: 1.0225x; 1.0225x over previous
import functools

import jax
import jax.numpy as jnp
from jax import lax
from jax.experimental import pallas as pl
from jax.experimental.pallas import tpu as pltpu
from jax.experimental.pallas import tpu_sc as plsc

D_M = 128
T_LEN = 200
N_WORKERS = 32
LANES = 16
CHUNKS_PER_T = D_M // LANES
PAIR = 2


def _sc_body(x_hbm, pe_hbm, out_hbm, pe_v, buf0, buf1,
             psem, isem0, isem1, osem0, osem1):
    nc = lax.axis_size("c")
    wid = lax.axis_index("s") * nc + lax.axis_index("c")
    b_per_w = out_hbm.shape[0] // N_WORKERS
    base = wid * b_per_w
    n_pairs = b_per_w // PAIR

    pe_h = pltpu.async_copy(pe_hbm, pe_v, psem)

    bufs = (buf0, buf1)
    isems = (isem0, isem1)
    osems = (osem0, osem1)
    in_h = [None, None]
    out_h = [None, None]

    in_h[0] = pltpu.async_copy(
        x_hbm.at[pl.ds(base, PAIR)], bufs[0], isems[0])
    pe_h.wait()
    for p in range(n_pairs):
        cur = p % 2
        nxt = 1 - cur
        if p + 1 < n_pairs:
            if out_h[nxt] is not None:
                out_h[nxt].wait()
            in_h[nxt] = pltpu.async_copy(
                x_hbm.at[pl.ds(base + (p + 1) * PAIR, PAIR)],
                bufs[nxt], isems[nxt])
        in_h[cur].wait()
        buf = bufs[cur]

        def add_body(t, _, buf=buf):
            for c in range(CHUNKS_PER_T):
                s = pl.ds(c * LANES, LANES)
                pv = pe_v[t, s]
                buf[0, t, s] = buf[0, t, s] + pv
                buf[1, t, s] = buf[1, t, s] + pv
            return 0

        lax.fori_loop(0, T_LEN, add_body, 0)
        out_h[cur] = pltpu.async_copy(
            buf, out_hbm.at[pl.ds(base + p * PAIR, PAIR)], osems[cur])
    for h in out_h:
        if h is not None:
            h.wait()


def _sc_add(x, pe_t):
    B = x.shape[0]
    mesh = plsc.VectorSubcoreMesh(core_axis_name="c", subcore_axis_name="s")
    f = functools.partial(
        pl.kernel,
        out_type=jax.ShapeDtypeStruct((B, T_LEN, D_M), jnp.float32),
        mesh=mesh,
        scratch_types=[
            pltpu.VMEM((T_LEN, D_M), jnp.float32),
            pltpu.VMEM((PAIR, T_LEN, D_M), jnp.float32),
            pltpu.VMEM((PAIR, T_LEN, D_M), jnp.float32),
            pltpu.SemaphoreType.DMA,
            pltpu.SemaphoreType.DMA,
            pltpu.SemaphoreType.DMA,
            pltpu.SemaphoreType.DMA,
            pltpu.SemaphoreType.DMA,
        ],
    )(_sc_body)
    return f(x, pe_t)


def kernel(x, pe):
    B, T, D = x.shape
    return _sc_add(x, pe[:T])
